# baseline (speedup 1.0000x reference)
import functools

import jax
import jax.numpy as jnp
from jax import lax
from jax.experimental import pallas as pl
from jax.experimental.pallas import tpu as pltpu
from jax.experimental.pallas import tpu_sc as plsc

B = 4096
L = 200
D = 64
N = B * L
NC = 2
NS = 16
NW = NC * NS
BG = B // NW
CH_L = 2
N_CHUNKS = L // CH_L
LANES = 16
DJ = D // LANES
LP = 104


def _body(idx_hbm, tok_hbm, pos_hbm, out_hbm,
          idx_a, idx_b, rows_a, rows_b, out_a, out_b, pos_v,
          gsem_a, gsem_b, osem_a, osem_b, isem_a, isem_b):
    cid = lax.axis_index("c")
    sid = lax.axis_index("s")
    wid = sid * NC + cid
    b0 = wid * BG

    pltpu.sync_copy(pos_hbm, pos_v)

    def ifire(c, idx_v, isem):
        lt = c >> 2
        ll0 = (c & 3) * CH_L
        pltpu.async_copy(idx_hbm.at[lt, wid, pl.ds(ll0, CH_L)], idx_v, isem)

    def gfire(c, idx_v, rows_v, isem, gsem):
        lt = c >> 2
        ll0 = (c & 3) * CH_L
        pltpu.make_async_copy(
            idx_hbm.at[lt, wid, pl.ds(ll0, CH_L)], idx_v, isem
        ).wait()
        pltpu.async_copy(tok_hbm.at[idx_v.at[0]], rows_v.at[0], gsem)
        pltpu.async_copy(tok_hbm.at[idx_v.at[1]], rows_v.at[1], gsem)

    def fire(c, idx_v, rows_v, isem, gsem):
        ifire(c, idx_v, isem)
        gfire(c, idx_v, rows_v, isem, gsem)

    def gdrain(idx_v, rows_v, gsem):
        pltpu.make_async_copy(tok_hbm.at[idx_v.at[0]], rows_v.at[0], gsem).wait()
        pltpu.make_async_copy(tok_hbm.at[idx_v.at[1]], rows_v.at[1], gsem).wait()

    def repack(c, rows_v, out_v):
        l0 = c * CH_L
        pvs = tuple(
            pos_v[l0 + p, pl.ds(j * LANES, LANES)]
            for p in range(CH_L)
            for j in range(DJ)
        )

        def b_body(bi, carry):
            for p in range(CH_L):
                for j in range(DJ):
                    out_v[bi, pl.ds(p * D + j * LANES, LANES)] = (
                        rows_v[p, bi, pl.ds(j * LANES, LANES)]
                        + carry[p * DJ + j]
                    )
            return carry

        lax.fori_loop(0, BG, b_body, pvs)

    def ofire(c, out_v, osem):
        pltpu.async_copy(out_v, out_hbm.at[pl.ds(b0, BG), c], osem)

    def odrain(c, out_v, osem):
        pltpu.make_async_copy(
            out_v, out_hbm.at[pl.ds(b0, BG), c], osem
        ).wait()

    fire(0, idx_a, rows_a, isem_a, gsem_a)
    fire(1, idx_b, rows_b, isem_b, gsem_b)
    gdrain(idx_a, rows_a, gsem_a)
    ifire(2, idx_a, isem_a)
    repack(0, rows_a, out_a)
    ofire(0, out_a, osem_a)
    gfire(2, idx_a, rows_a, isem_a, gsem_a)
    gdrain(idx_b, rows_b, gsem_b)
    ifire(3, idx_b, isem_b)
    repack(1, rows_b, out_b)
    ofire(1, out_b, osem_b)
    gfire(3, idx_b, rows_b, isem_b, gsem_b)

    def pair_body(cc, carry):
        c0 = 2 * cc
        gdrain(idx_a, rows_a, gsem_a)
        ifire(c0 + 2, idx_a, isem_a)
        odrain(c0, out_a, osem_a)
        repack(c0, rows_a, out_a)
        ofire(c0, out_a, osem_a)
        gfire(c0 + 2, idx_a, rows_a, isem_a, gsem_a)
        gdrain(idx_b, rows_b, gsem_b)
        ifire(c0 + 3, idx_b, isem_b)
        odrain(c0 + 1, out_b, osem_b)
        repack(c0 + 1, rows_b, out_b)
        ofire(c0 + 1, out_b, osem_b)
        gfire(c0 + 3, idx_b, rows_b, isem_b, gsem_b)
        return carry

    lax.fori_loop(1, N_CHUNKS // 2 - 1, pair_body, 0)

    c_last = N_CHUNKS - 2
    gdrain(idx_a, rows_a, gsem_a)
    odrain(c_last, out_a, osem_a)
    repack(c_last, rows_a, out_a)
    ofire(c_last, out_a, osem_a)
    gdrain(idx_b, rows_b, gsem_b)
    odrain(c_last + 1, out_b, osem_b)
    repack(c_last + 1, rows_b, out_b)
    ofire(c_last + 1, out_b, osem_b)
    odrain(c_last, out_a, osem_a)
    odrain(c_last + 1, out_b, osem_b)


@jax.jit
def _sc_embed(idx4, token_table, pos_table):
    mesh = plsc.VectorSubcoreMesh(
        core_axis_name="c", subcore_axis_name="s", num_cores=NC, num_subcores=NS
    )
    return pl.kernel(
        _body,
        out_type=jax.ShapeDtypeStruct((B, LP, 128), jnp.float32),
        mesh=mesh,
        scratch_types=[
            pltpu.VMEM((CH_L, BG), jnp.int32),
            pltpu.VMEM((CH_L, BG), jnp.int32),
            pltpu.VMEM((CH_L, BG, D), jnp.float32),
            pltpu.VMEM((CH_L, BG, D), jnp.float32),
            pltpu.VMEM((BG, 128), jnp.float32),
            pltpu.VMEM((BG, 128), jnp.float32),
            pltpu.VMEM((L, D), jnp.float32),
            pltpu.SemaphoreType.DMA,
            pltpu.SemaphoreType.DMA,
            pltpu.SemaphoreType.DMA,
            pltpu.SemaphoreType.DMA,
            pltpu.SemaphoreType.DMA,
            pltpu.SemaphoreType.DMA,
        ],
        compiler_params=pltpu.CompilerParams(use_tc_tiling_on_sc=False),
    )(idx4, token_table, pos_table)


def kernel(inputs, token_table, pos_table):
    idx4 = (
        inputs.astype(jnp.int32)
        .reshape(NW, BG, L // 8, 8)
        .transpose(2, 0, 3, 1)
    )
    out = _sc_embed(idx4, token_table, pos_table)
    return out[:, : L // 2, :].reshape(B, L, D)


# speedup vs baseline: 1.8674x; 1.8674x over previous
import functools

import jax
import jax.numpy as jnp
from jax import lax
from jax.experimental import pallas as pl
from jax.experimental.pallas import tpu as pltpu
from jax.experimental.pallas import tpu_sc as plsc

B = 4096
L = 200
D = 64
N = B * L
NC = 2
NS = 16
NW = NC * NS
ROWS_PER_W = N // NW
SEQ_PER_W = ROWS_PER_W // L
CH_SEQ = 2
CH_ROWS = CH_SEQ * L
N_CHUNKS = SEQ_PER_W // CH_SEQ
G = 100
N_GATHER = CH_ROWS // G
LANES = 16
DJ = D // LANES


def _body(idx_hbm, tok_hbm, pos_hbm, out_hbm,
          idx_a, idx_b, rows_a, rows_b, out_a, out_b, pos_v,
          gsem_a, gsem_b, osem_a, osem_b, isem_a, isem_b):
    cid = lax.axis_index("c")
    sid = lax.axis_index("s")
    wid = sid * NC + cid

    pltpu.sync_copy(pos_hbm, pos_v)

    def ifire(c, idx_v, isem):
        idx_row = wid * (ROWS_PER_W // G) + c * N_GATHER
        pltpu.async_copy(idx_hbm.at[pl.ds(idx_row, N_GATHER)], idx_v, isem)

    def gfire(c, idx_v, rows_v, isem, gsem):
        idx_row = wid * (ROWS_PER_W // G) + c * N_GATHER
        pltpu.make_async_copy(
            idx_hbm.at[pl.ds(idx_row, N_GATHER)], idx_v, isem
        ).wait()
        for u in range(N_GATHER):
            s, h = divmod(u, L // G)
            pltpu.async_copy(
                tok_hbm.at[idx_v.at[u]],
                rows_v.at[s, pl.ds(h * G, G)],
                gsem,
            )

    def fire(c, idx_v, rows_v, isem, gsem):
        ifire(c, idx_v, isem)
        gfire(c, idx_v, rows_v, isem, gsem)

    def gdrain(idx_v, rows_v, gsem):
        for u in range(N_GATHER):
            s, h = divmod(u, L // G)
            pltpu.make_async_copy(
                tok_hbm.at[idx_v.at[u]],
                rows_v.at[s, pl.ds(h * G, G)],
                gsem,
            ).wait()

    def repack(rows_v, out_v):
        @plsc.parallel_loop(0, L // 2, unroll=2)
        def l_body(lh):
            for par in range(2):
                l = 2 * lh + par
                for j in range(DJ):
                    pv = pos_v[l, pl.ds(j * LANES, LANES)]
                    col = par * D + j * LANES
                    for s in range(CH_SEQ):
                        out_v[s, lh, pl.ds(col, LANES)] = (
                            rows_v[s, l, pl.ds(j * LANES, LANES)] + pv
                        )

    def ofire(c, out_v, osem):
        seq_base = wid * SEQ_PER_W + c * CH_SEQ
        pltpu.async_copy(
            out_v, out_hbm.at[pl.ds(seq_base, CH_SEQ), pl.ds(0, L // 2)], osem
        )

    def odrain(c, out_v, osem):
        seq_base = wid * SEQ_PER_W + c * CH_SEQ
        pltpu.make_async_copy(
            out_v, out_hbm.at[pl.ds(seq_base, CH_SEQ), pl.ds(0, L // 2)], osem
        ).wait()

    fire(0, idx_a, rows_a, isem_a, gsem_a)
    fire(1, idx_b, rows_b, isem_b, gsem_b)
    gdrain(idx_a, rows_a, gsem_a)
    ifire(2, idx_a, isem_a)
    repack(rows_a, out_a)
    ofire(0, out_a, osem_a)
    gfire(2, idx_a, rows_a, isem_a, gsem_a)
    gdrain(idx_b, rows_b, gsem_b)
    ifire(3, idx_b, isem_b)
    repack(rows_b, out_b)
    ofire(1, out_b, osem_b)
    gfire(3, idx_b, rows_b, isem_b, gsem_b)

    def pair_body(cc, carry):
        c0 = 2 * cc
        gdrain(idx_a, rows_a, gsem_a)
        ifire(c0 + 2, idx_a, isem_a)
        odrain(c0, out_a, osem_a)
        repack(rows_a, out_a)
        ofire(c0, out_a, osem_a)
        gfire(c0 + 2, idx_a, rows_a, isem_a, gsem_a)
        gdrain(idx_b, rows_b, gsem_b)
        ifire(c0 + 3, idx_b, isem_b)
        odrain(c0 + 1, out_b, osem_b)
        repack(rows_b, out_b)
        ofire(c0 + 1, out_b, osem_b)
        gfire(c0 + 3, idx_b, rows_b, isem_b, gsem_b)
        return carry

    lax.fori_loop(1, N_CHUNKS // 2 - 1, pair_body, 0)

    c_last = N_CHUNKS - 2
    gdrain(idx_a, rows_a, gsem_a)
    odrain(c_last, out_a, osem_a)
    repack(rows_a, out_a)
    ofire(c_last, out_a, osem_a)
    gdrain(idx_b, rows_b, gsem_b)
    odrain(c_last + 1, out_b, osem_b)
    repack(rows_b, out_b)
    ofire(c_last + 1, out_b, osem_b)
    odrain(c_last, out_a, osem_a)
    odrain(c_last + 1, out_b, osem_b)


@jax.jit
def _sc_embed(idx2d, token_table, pos_table):
    mesh = plsc.VectorSubcoreMesh(
        core_axis_name="c", subcore_axis_name="s", num_cores=NC, num_subcores=NS
    )
    return pl.kernel(
        _body,
        out_type=jax.ShapeDtypeStruct((B, 104, 128), jnp.float32),
        mesh=mesh,
        scratch_types=[
            pltpu.VMEM((N_GATHER, G), jnp.int32),
            pltpu.VMEM((N_GATHER, G), jnp.int32),
            pltpu.VMEM((CH_SEQ, L, D), jnp.float32),
            pltpu.VMEM((CH_SEQ, L, D), jnp.float32),
            pltpu.VMEM((CH_SEQ, L // 2, 128), jnp.float32),
            pltpu.VMEM((CH_SEQ, L // 2, 128), jnp.float32),
            pltpu.VMEM((L, D), jnp.float32),
            pltpu.SemaphoreType.DMA,
            pltpu.SemaphoreType.DMA,
            pltpu.SemaphoreType.DMA,
            pltpu.SemaphoreType.DMA,
            pltpu.SemaphoreType.DMA,
            pltpu.SemaphoreType.DMA,
        ],
        compiler_params=pltpu.CompilerParams(use_tc_tiling_on_sc=False),
    )(idx2d, token_table, pos_table)


def kernel(inputs, token_table, pos_table):
    idx2d = inputs.reshape(N // G, G).astype(jnp.int32)
    out = _sc_embed(idx2d, token_table, pos_table)
    return out[:, : L // 2, :].reshape(B, L, D)
